# Initial kernel scaffold; baseline (speedup 1.0000x reference)
import jax
import jax.numpy as jnp
from jax.experimental import pallas as pl


def kernel(x, edge_index, edge_attr, batch, Ws, bs, Wes, bes, fc_W, fc_b):
    raise NotImplementedError("write your pallas kernel here")


# speedup vs baseline: 3.7672x; 3.7672x over previous
import functools

import jax
import jax.numpy as jnp
from jax import lax
from jax.experimental import pallas as pl
from jax.experimental.pallas import tpu as pltpu
from jax.experimental.pallas import tpu_sc as plsc

G = 64

NC = 2
NS = 16
K = 128
ZR = 128


def _ceil_to(a, m):
    return (a + m - 1) // m * m


def _sc_segsum_call(h, src, dst, n_acc, gather):
    e_pad = dst.shape[0]
    dw = h.shape[1]
    epw = e_pad // (NC * NS)
    steps = epw // K
    rows_per_sub = n_acc // NS
    zsteps = rows_per_sub // ZR
    mesh = plsc.VectorSubcoreMesh(core_axis_name="c", subcore_axis_name="s")

    zrows = jnp.zeros((ZR, dw), dtype=jnp.float32)

    scratch = [
        pltpu.VMEM((K,), jnp.int32),
        pltpu.VMEM((K,), jnp.int32),
        pltpu.VMEM((K, dw), jnp.float32),
        pltpu.VMEM_SHARED((n_acc, dw), jnp.float32),
    ]

    @functools.partial(
        pl.kernel,
        out_type=jax.ShapeDtypeStruct((NC, n_acc, dw), jnp.float32),
        mesh=mesh,
        scratch_types=scratch,
    )
    def seg_kernel(h_hbm, src_hbm, dst_hbm, z_hbm, out_hbm, idx_s, idx_d,
                   rows, acc):
        cid = lax.axis_index("c")
        sid = lax.axis_index("s")
        wid = cid * NS + sid

        pltpu.sync_copy(z_hbm, rows)
        row0 = sid * rows_per_sub

        @pl.loop(0, zsteps)
        def _(t):
            pltpu.sync_copy(rows, acc.at[pl.ds(row0 + t * ZR, ZR)])

        plsc.subcore_barrier()

        base = wid * epw

        @pl.loop(0, steps)
        def _(t):
            off = base + t * K
            pltpu.sync_copy(dst_hbm.at[pl.ds(off, K)], idx_d)
            if gather:
                pltpu.sync_copy(src_hbm.at[pl.ds(off, K)], idx_s)
                pltpu.sync_copy(h_hbm.at[idx_s], rows)
            else:
                pltpu.sync_copy(h_hbm.at[pl.ds(off, K)], rows)
            pltpu.sync_copy(rows, acc.at[idx_d], add=True)

        plsc.subcore_barrier()

        @pl.loop(0, zsteps)
        def _(t):
            pltpu.sync_copy(acc.at[pl.ds(row0 + t * ZR, ZR)],
                            out_hbm.at[cid].at[pl.ds(row0 + t * ZR, ZR)])

    return seg_kernel(h, src, dst, zrows)


def _layer_body(h_ref, p_ref, e_ref, wt_ref, w_ref, b_ref, o_ref):
    pre = h_ref[...] + p_ref[0] + p_ref[1]
    ea = e_ref[0] + e_ref[1]
    pre = pre + jnp.dot(ea, wt_ref[...], preferred_element_type=jnp.float32)
    z = jnp.dot(pre, w_ref[...], preferred_element_type=jnp.float32)
    o_ref[...] = jnp.maximum(z + b_ref[...], 0.0)


def _tc_layer(h, parts, ea_parts, wtil, w, b2d, blk):
    n, d = h.shape
    de16 = ea_parts.shape[2]
    grid = (n // blk,)
    return pl.pallas_call(
        _layer_body,
        grid=grid,
        in_specs=[
            pl.BlockSpec((blk, d), lambda i: (i, 0)),
            pl.BlockSpec((NC, blk, d), lambda i: (0, i, 0)),
            pl.BlockSpec((NC, blk, de16), lambda i: (0, i, 0)),
            pl.BlockSpec((de16, d), lambda i: (0, 0)),
            pl.BlockSpec((d, d), lambda i: (0, 0)),
            pl.BlockSpec((1, d), lambda i: (0, 0)),
        ],
        out_specs=pl.BlockSpec((blk, d), lambda i: (i, 0)),
        out_shape=jax.ShapeDtypeStruct((n, d), jnp.float32),
    )(h, parts, ea_parts, wtil, w, b2d)


def _pool_body(h_ref, b_ref, fw_ref, fb_ref, o_ref):
    n = h_ref.shape[0]
    seg = lax.broadcasted_iota(jnp.int32, (n, G), 1)
    onehot = (b_ref[...] == seg).astype(jnp.float32)
    sums = lax.dot_general(onehot, h_ref[...], (((0,), (0,)), ((), ())),
                           preferred_element_type=jnp.float32)
    counts = jnp.sum(onehot, axis=0)[:, None]
    pooled = sums / jnp.maximum(counts, 1.0)
    logits = jnp.dot(pooled, fw_ref[...],
                     preferred_element_type=jnp.float32) + fb_ref[...]
    m = jnp.max(logits, axis=1, keepdims=True)
    ex = jnp.exp(logits - m)
    o_ref[...] = ex / jnp.sum(ex, axis=1, keepdims=True)


def _tc_pool(h, batch2d, fc_w, fc_b2d):
    n, d = h.shape
    out = fc_w.shape[1]
    return pl.pallas_call(
        _pool_body,
        in_specs=[
            pl.BlockSpec((n, d), lambda: (0, 0)),
            pl.BlockSpec((n, 1), lambda: (0, 0)),
            pl.BlockSpec((d, out), lambda: (0, 0)),
            pl.BlockSpec((1, out), lambda: (0, 0)),
        ],
        out_specs=pl.BlockSpec((G, out), lambda: (0, 0)),
        out_shape=jax.ShapeDtypeStruct((G, out), jnp.float32),
    )(h, batch2d, fc_w, fc_b2d)


@jax.jit
def kernel(x, edge_index, edge_attr, batch, Ws, bs, Wes, bes, fc_W, fc_b):
    n, d = x.shape
    e = edge_index.shape[1]
    nl, de, _ = Wes.shape

    e_pad = _ceil_to(e, NC * NS * K)
    n_acc = _ceil_to(n, NS * ZR)

    pad = e_pad - e
    src = jnp.concatenate([edge_index[0], jnp.zeros((pad,), jnp.int32)])
    dst = jnp.concatenate([edge_index[1], jnp.full((pad,), n, jnp.int32)])

    de16 = 16
    ea_aug = jnp.concatenate(
        [edge_attr.astype(jnp.float32),
         jnp.ones((e, 1), jnp.float32),
         jnp.zeros((e, de16 - de - 1), jnp.float32)], axis=1)
    ea_aug = jnp.concatenate([ea_aug, jnp.zeros((pad, de16), jnp.float32)])

    wtil = jnp.concatenate(
        [Wes.astype(jnp.float32),
         bes.astype(jnp.float32)[:, None, :],
         jnp.zeros((nl, de16 - de - 1, d), jnp.float32)], axis=1)

    ea_parts = _sc_segsum_call(ea_aug, src, dst, n_acc, gather=False)

    b2d = bs.astype(jnp.float32)[:, None, :]
    h = x.astype(jnp.float32)
    for l in range(nl):
        parts = _sc_segsum_call(h, src, dst, n_acc, gather=True)
        h = _tc_layer(h, parts[:, :n], ea_parts[:, :n], wtil[l],
                      Ws[l].astype(jnp.float32), b2d[l], blk=2000)

    batch2d = batch.astype(jnp.int32)[:, None]
    return _tc_pool(h, batch2d, fc_W.astype(jnp.float32),
                    fc_b.astype(jnp.float32)[None, :])
